# Initial kernel scaffold; baseline (speedup 1.0000x reference)
import jax
import jax.numpy as jnp
from jax.experimental import pallas as pl


def kernel(x, edge_index, W_l, b_l, W_r):
    raise NotImplementedError("write your pallas kernel here")


# speedup vs baseline: 7.9470x; 7.9470x over previous
import functools

import jax
import jax.numpy as jnp
from jax import lax
from jax.experimental import pallas as pl
from jax.experimental.pallas import tpu as pltpu
from jax.experimental.pallas import tpu_sc as plsc

N = 10000
NP = 10240
E = 320000
DIN = 128
DOUT = 256

NC = 2
NS = 16
NW = NC * NS
EPW = E // NW
CH = 40
NCHUNK = EPW // CH
RPT = NP // NS


def _sc_aggregate(x, src, dst, zrows, zcnt):
    mesh = plsc.VectorSubcoreMesh(
        core_axis_name="c", subcore_axis_name="s", num_cores=NC, num_subcores=NS
    )

    @functools.partial(
        pl.kernel,
        out_type=(
            jax.ShapeDtypeStruct((NC, NP, DIN), jnp.float32),
            jax.ShapeDtypeStruct((NC, NP), jnp.float32),
        ),
        mesh=mesh,
        scratch_types=(
            pltpu.VMEM_SHARED((NP, DIN), jnp.float32),
            pltpu.VMEM_SHARED((NP,), jnp.float32),
            pltpu.VMEM((2, CH), jnp.int32),
            pltpu.VMEM((2, CH), jnp.int32),
            pltpu.VMEM((2, CH, DIN), jnp.float32),
            pltpu.VMEM((48,), jnp.float32),
            pltpu.SemaphoreType.DMA((2,)),
            pltpu.SemaphoreType.DMA((2,)),
        ),
    )
    def k(x_hbm, src_hbm, dst_hbm, zr_hbm, zc_hbm, psum_hbm, pcnt_hbm,
          acc, cnt, srcix, dstix, rows, ones, gsem, isem):
        cid = lax.axis_index("c")
        sid = lax.axis_index("s")
        wid = sid * NC + cid
        base = wid * EPW

        r0 = pl.multiple_of(sid * RPT, 128)
        pltpu.sync_copy(zr_hbm.at[pl.ds(r0, RPT)], acc.at[pl.ds(r0, RPT)])
        pltpu.sync_copy(zc_hbm.at[pl.ds(r0, RPT)], cnt.at[pl.ds(r0, RPT)])

        for i in range(3):
            ones[pl.ds(i * 16, 16)] = jnp.ones((16,), jnp.float32)

        plsc.subcore_barrier()

        def load_idx(j, slot):
            pltpu.async_copy(src_hbm.at[pl.ds(base + j * CH, CH)],
                             srcix.at[slot], isem.at[slot])
            pltpu.async_copy(dst_hbm.at[pl.ds(base + j * CH, CH)],
                             dstix.at[slot], isem.at[slot])

        def wait_idx(slot):
            pltpu.make_async_copy(src_hbm.at[pl.ds(0, CH)],
                                  srcix.at[slot], isem.at[slot]).wait()
            pltpu.make_async_copy(dst_hbm.at[pl.ds(0, CH)],
                                  dstix.at[slot], isem.at[slot]).wait()

        load_idx(0, 0)
        load_idx(1, 1)
        wait_idx(0)
        pltpu.async_copy(x_hbm.at[srcix.at[0]], rows.at[0], gsem.at[0])

        def body(j, _):
            jm2 = lax.rem(j, 2)
            nm2 = 1 - jm2
            pltpu.make_async_copy(
                x_hbm.at[srcix.at[jm2]], rows.at[jm2], gsem.at[jm2]
            ).wait()

            @pl.when(j + 1 < NCHUNK)
            def _():
                wait_idx(nm2)
                pltpu.async_copy(
                    x_hbm.at[srcix.at[nm2]], rows.at[nm2], gsem.at[nm2]
                )

            pltpu.sync_copy(rows.at[jm2], acc.at[dstix.at[jm2]], add=True)
            pltpu.sync_copy(ones.at[pl.ds(0, CH)], cnt.at[dstix.at[jm2]], add=True)

            @pl.when(j + 2 < NCHUNK)
            def _():
                load_idx(j + 2, jm2)

            return ()

        lax.fori_loop(0, NCHUNK, body, (), unroll=False)

        plsc.subcore_barrier()

        pltpu.sync_copy(acc.at[pl.ds(r0, RPT)], psum_hbm.at[cid, pl.ds(r0, RPT)])
        pltpu.sync_copy(cnt.at[pl.ds(r0, RPT)], pcnt_hbm.at[cid, pl.ds(r0, RPT)])

    return k(x, src, dst, zrows, zcnt)


BR = 1024


def _dense_body(ps_ref, pc_ref, x_ref, wl_ref, bl_ref, wr_ref, o_ref):
    s = ps_ref[0] + ps_ref[1]
    c = pc_ref[0] + pc_ref[1]
    inv = 1.0 / jnp.clip(c, 1.0, None)
    mean = s * inv[:, None]
    hi = jax.lax.Precision.HIGHEST
    o_ref[...] = (
        jnp.dot(mean, wl_ref[...], precision=hi)
        + jnp.dot(x_ref[...], wr_ref[...], precision=hi)
        + bl_ref[...]
    )


def _dense(psum, pcnt, x, wl_t, bl, wr_t):
    grid = (pl.cdiv(N, BR),)
    return pl.pallas_call(
        _dense_body,
        grid=grid,
        in_specs=[
            pl.BlockSpec((NC, BR, DIN), lambda i: (0, i, 0)),
            pl.BlockSpec((NC, BR), lambda i: (0, i)),
            pl.BlockSpec((BR, DIN), lambda i: (i, 0)),
            pl.BlockSpec((DIN, DOUT), lambda i: (0, 0)),
            pl.BlockSpec((1, DOUT), lambda i: (0, 0)),
            pl.BlockSpec((DIN, DOUT), lambda i: (0, 0)),
        ],
        out_specs=pl.BlockSpec((BR, DOUT), lambda i: (i, 0)),
        out_shape=jax.ShapeDtypeStruct((N, DOUT), jnp.float32),
    )(psum, pcnt, x, wl_t, bl, wr_t)


def kernel(x, edge_index, W_l, b_l, W_r):
    src = edge_index[0]
    dst = edge_index[1]
    zr = jnp.zeros((NP, DIN), jnp.float32)
    zc = jnp.zeros((NP,), jnp.float32)
    psum, pcnt = _sc_aggregate(x, src, dst, zr, zc)
    return _dense(psum, pcnt, x, W_l.T, b_l.reshape(1, DOUT), W_r.T)


# speedup vs baseline: 10.7934x; 1.3582x over previous
import functools

import jax
import jax.numpy as jnp
from jax import lax
from jax.experimental import pallas as pl
from jax.experimental.pallas import tpu as pltpu
from jax.experimental.pallas import tpu_sc as plsc

N = 10000
NP = 10240
E = 320000
DIN = 128
DOUT = 256

NC = 2
NS = 16
NW = NC * NS
EPW = E // NW
CH = 80
NCHUNK = EPW // CH
RPT = NP // NS


def _sc_aggregate(x, src, dst, zrows, zcnt):
    mesh = plsc.VectorSubcoreMesh(
        core_axis_name="c", subcore_axis_name="s", num_cores=NC, num_subcores=NS
    )

    @functools.partial(
        pl.kernel,
        out_type=(
            jax.ShapeDtypeStruct((NC, NP, DIN), jnp.float32),
            jax.ShapeDtypeStruct((NC, NP), jnp.float32),
        ),
        mesh=mesh,
        scratch_types=(
            pltpu.VMEM_SHARED((NP, DIN), jnp.float32),
            pltpu.VMEM_SHARED((NP,), jnp.float32),
            pltpu.VMEM((4, CH), jnp.int32),
            pltpu.VMEM((4, CH), jnp.int32),
            pltpu.VMEM((3, CH, DIN), jnp.float32),
            pltpu.VMEM((CH,), jnp.float32),
            pltpu.SemaphoreType.DMA((3,)),
            pltpu.SemaphoreType.DMA((3,)),
            pltpu.SemaphoreType.DMA((4,)),
        ),
    )
    def k(x_hbm, src_hbm, dst_hbm, zr_hbm, zc_hbm, psum_hbm, pcnt_hbm,
          acc, cnt, srcix, dstix, rows, ones, gsem, ssem, isem):
        cid = lax.axis_index("c")
        sid = lax.axis_index("s")
        wid = sid * NC + cid
        base = wid * EPW

        r0 = pl.multiple_of(sid * RPT, 128)
        pltpu.sync_copy(zr_hbm.at[pl.ds(r0, RPT)], acc.at[pl.ds(r0, RPT)])
        pltpu.sync_copy(zc_hbm.at[pl.ds(r0, RPT)], cnt.at[pl.ds(r0, RPT)])

        for i in range(CH // 16):
            ones[pl.ds(i * 16, 16)] = jnp.ones((16,), jnp.float32)

        plsc.subcore_barrier()

        def load_idx(j, slot):
            pltpu.async_copy(src_hbm.at[pl.ds(base + j * CH, CH)],
                             srcix.at[slot], isem.at[slot])
            pltpu.async_copy(dst_hbm.at[pl.ds(base + j * CH, CH)],
                             dstix.at[slot], isem.at[slot])

        def wait_idx(slot):
            pltpu.make_async_copy(src_hbm.at[pl.ds(0, CH)],
                                  srcix.at[slot], isem.at[slot]).wait()
            pltpu.make_async_copy(dst_hbm.at[pl.ds(0, CH)],
                                  dstix.at[slot], isem.at[slot]).wait()

        def wait_scatter(slot):
            pltpu.make_async_copy(rows.at[slot], acc.at[dstix.at[0]],
                                  ssem.at[slot]).wait()
            pltpu.make_async_copy(ones, cnt.at[dstix.at[0]],
                                  ssem.at[slot]).wait()

        load_idx(0, 0)
        load_idx(1, 1)
        wait_idx(0)
        pltpu.async_copy(x_hbm.at[srcix.at[0]], rows.at[0], gsem.at[0])

        def body(j, _):
            b3 = lax.rem(j, 3)
            s4 = lax.rem(j, 4)
            pltpu.make_async_copy(
                x_hbm.at[srcix.at[s4]], rows.at[b3], gsem.at[b3]
            ).wait()

            @pl.when(j + 1 < NCHUNK)
            def _():
                nb3 = lax.rem(j + 1, 3)
                ns4 = lax.rem(j + 1, 4)

                @pl.when(j >= 2)
                def _():
                    wait_scatter(nb3)

                wait_idx(ns4)
                pltpu.async_copy(
                    x_hbm.at[srcix.at[ns4]], rows.at[nb3], gsem.at[nb3]
                )

            pltpu.async_copy(rows.at[b3], acc.at[dstix.at[s4]],
                             ssem.at[b3], add=True)
            pltpu.async_copy(ones, cnt.at[dstix.at[s4]],
                             ssem.at[b3], add=True)

            @pl.when(j + 2 < NCHUNK)
            def _():
                load_idx(j + 2, lax.rem(j + 2, 4))

            return ()

        lax.fori_loop(0, NCHUNK, body, (), unroll=False)

        wait_scatter((NCHUNK - 3) % 3)
        wait_scatter((NCHUNK - 2) % 3)
        wait_scatter((NCHUNK - 1) % 3)

        plsc.subcore_barrier()

        pltpu.sync_copy(acc.at[pl.ds(r0, RPT)], psum_hbm.at[cid, pl.ds(r0, RPT)])
        pltpu.sync_copy(cnt.at[pl.ds(r0, RPT)], pcnt_hbm.at[cid, pl.ds(r0, RPT)])

    return k(x, src, dst, zrows, zcnt)


BR = 1024


def _dense_body(ps_ref, pc_ref, x_ref, wl_ref, bl_ref, wr_ref, o_ref):
    s = ps_ref[0] + ps_ref[1]
    c = pc_ref[0] + pc_ref[1]
    inv = 1.0 / jnp.clip(c, 1.0, None)
    mean = s * inv[:, None]
    hi = jax.lax.Precision.HIGHEST
    o_ref[...] = (
        jnp.dot(mean, wl_ref[...], precision=hi)
        + jnp.dot(x_ref[...], wr_ref[...], precision=hi)
        + bl_ref[...]
    )


def _dense(psum, pcnt, x, wl_t, bl, wr_t):
    grid = (pl.cdiv(N, BR),)
    return pl.pallas_call(
        _dense_body,
        grid=grid,
        in_specs=[
            pl.BlockSpec((NC, BR, DIN), lambda i: (0, i, 0)),
            pl.BlockSpec((NC, BR), lambda i: (0, i)),
            pl.BlockSpec((BR, DIN), lambda i: (i, 0)),
            pl.BlockSpec((DIN, DOUT), lambda i: (0, 0)),
            pl.BlockSpec((1, DOUT), lambda i: (0, 0)),
            pl.BlockSpec((DIN, DOUT), lambda i: (0, 0)),
        ],
        out_specs=pl.BlockSpec((BR, DOUT), lambda i: (i, 0)),
        out_shape=jax.ShapeDtypeStruct((N, DOUT), jnp.float32),
    )(psum, pcnt, x, wl_t, bl, wr_t)


def kernel(x, edge_index, W_l, b_l, W_r):
    src = edge_index[0]
    dst = edge_index[1]
    zr = jnp.zeros((NP, DIN), jnp.float32)
    zc = jnp.zeros((NP,), jnp.float32)
    psum, pcnt = _sc_aggregate(x, src, dst, zr, zc)
    return _dense(psum, pcnt, x, W_l.T, b_l.reshape(1, DOUT), W_r.T)


# speedup vs baseline: 13.3174x; 1.2338x over previous
import functools

import jax
import jax.numpy as jnp
from jax import lax
from jax.experimental import pallas as pl
from jax.experimental.pallas import tpu as pltpu
from jax.experimental.pallas import tpu_sc as plsc

N = 10000
NP = 10240
E = 320000
DIN = 128
DOUT = 256

NC = 2
NS = 16
NW = NC * NS
EPW = E // NW
CH = 80
NCHUNK = EPW // CH
RPT = NP // NS


def _sc_aggregate(x, src, dst, zrows, zcnt):
    mesh = plsc.VectorSubcoreMesh(
        core_axis_name="c", subcore_axis_name="s", num_cores=NC, num_subcores=NS
    )

    @functools.partial(
        pl.kernel,
        out_type=(
            jax.ShapeDtypeStruct((NC, NP, DIN), jnp.float32),
            jax.ShapeDtypeStruct((NC, NP), jnp.float32),
        ),
        mesh=mesh,
        scratch_types=(
            pltpu.VMEM_SHARED((NP, DIN), jnp.float32),
            pltpu.VMEM_SHARED((NP,), jnp.float32),
            pltpu.VMEM((6, CH), jnp.int32),
            pltpu.VMEM((6, CH), jnp.int32),
            pltpu.VMEM((4, CH, DIN), jnp.float32),
            pltpu.VMEM((CH,), jnp.float32),
            pltpu.SemaphoreType.DMA((4,)),
            pltpu.SemaphoreType.DMA((4,)),
            pltpu.SemaphoreType.DMA((6,)),
        ),
    )
    def k(x_hbm, src_hbm, dst_hbm, zr_hbm, zc_hbm, psum_hbm, pcnt_hbm,
          acc, cnt, srcix, dstix, rows, ones, gsem, ssem, isem):
        cid = lax.axis_index("c")
        sid = lax.axis_index("s")
        wid = sid * NC + cid
        base = wid * EPW

        r0 = pl.multiple_of(sid * RPT, 128)
        pltpu.sync_copy(zr_hbm.at[pl.ds(r0, RPT)], acc.at[pl.ds(r0, RPT)])
        pltpu.sync_copy(zc_hbm.at[pl.ds(r0, RPT)], cnt.at[pl.ds(r0, RPT)])

        for i in range(CH // 16):
            ones[pl.ds(i * 16, 16)] = jnp.ones((16,), jnp.float32)

        plsc.subcore_barrier()

        def load_idx(j, slot):
            pltpu.async_copy(src_hbm.at[pl.ds(base + j * CH, CH)],
                             srcix.at[slot], isem.at[slot])
            pltpu.async_copy(dst_hbm.at[pl.ds(base + j * CH, CH)],
                             dstix.at[slot], isem.at[slot])

        def wait_idx(slot):
            pltpu.make_async_copy(src_hbm.at[pl.ds(0, CH)],
                                  srcix.at[slot], isem.at[slot]).wait()
            pltpu.make_async_copy(dst_hbm.at[pl.ds(0, CH)],
                                  dstix.at[slot], isem.at[slot]).wait()

        def wait_scatter(slot):
            pltpu.make_async_copy(rows.at[slot], acc.at[dstix.at[0]],
                                  ssem.at[slot]).wait()
            pltpu.make_async_copy(ones, cnt.at[dstix.at[0]],
                                  ssem.at[slot]).wait()

        for j in range(4):
            load_idx(j, j)
        for j in range(2):
            wait_idx(j)
            pltpu.async_copy(x_hbm.at[srcix.at[j]], rows.at[j], gsem.at[j])

        def body(j, _):
            b4 = lax.rem(j, 4)
            s6 = lax.rem(j, 6)
            pltpu.make_async_copy(
                x_hbm.at[srcix.at[s6]], rows.at[b4], gsem.at[b4]
            ).wait()

            @pl.when(j + 2 < NCHUNK)
            def _():
                nb4 = lax.rem(j + 2, 4)
                ns6 = lax.rem(j + 2, 6)

                @pl.when(j >= 2)
                def _():
                    wait_scatter(nb4)

                wait_idx(ns6)
                pltpu.async_copy(
                    x_hbm.at[srcix.at[ns6]], rows.at[nb4], gsem.at[nb4]
                )

            pltpu.async_copy(rows.at[b4], acc.at[dstix.at[s6]],
                             ssem.at[b4], add=True)
            pltpu.async_copy(ones, cnt.at[dstix.at[s6]],
                             ssem.at[b4], add=True)

            @pl.when(j + 4 < NCHUNK)
            def _():
                load_idx(j + 4, lax.rem(j + 4, 6))

            return ()

        lax.fori_loop(0, NCHUNK, body, (), unroll=False)

        for j in range(NCHUNK - 4, NCHUNK):
            wait_scatter(j % 4)

        plsc.subcore_barrier()

        pltpu.sync_copy(acc.at[pl.ds(r0, RPT)], psum_hbm.at[cid, pl.ds(r0, RPT)])
        pltpu.sync_copy(cnt.at[pl.ds(r0, RPT)], pcnt_hbm.at[cid, pl.ds(r0, RPT)])

    return k(x, src, dst, zrows, zcnt)


BR = 1024


def _dense_body(ps_ref, pc_ref, x_ref, wl_ref, bl_ref, wr_ref, o_ref):
    s = ps_ref[0] + ps_ref[1]
    c = pc_ref[0] + pc_ref[1]
    inv = 1.0 / jnp.clip(c, 1.0, None)
    mean = s * inv[:, None]
    hi = jax.lax.Precision.HIGHEST
    o_ref[...] = (
        jnp.dot(mean, wl_ref[...], precision=hi)
        + jnp.dot(x_ref[...], wr_ref[...], precision=hi)
        + bl_ref[...]
    )


def _dense(psum, pcnt, x, wl_t, bl, wr_t):
    grid = (pl.cdiv(N, BR),)
    return pl.pallas_call(
        _dense_body,
        grid=grid,
        in_specs=[
            pl.BlockSpec((NC, BR, DIN), lambda i: (0, i, 0)),
            pl.BlockSpec((NC, BR), lambda i: (0, i)),
            pl.BlockSpec((BR, DIN), lambda i: (i, 0)),
            pl.BlockSpec((DIN, DOUT), lambda i: (0, 0)),
            pl.BlockSpec((1, DOUT), lambda i: (0, 0)),
            pl.BlockSpec((DIN, DOUT), lambda i: (0, 0)),
        ],
        out_specs=pl.BlockSpec((BR, DOUT), lambda i: (i, 0)),
        out_shape=jax.ShapeDtypeStruct((N, DOUT), jnp.float32),
    )(psum, pcnt, x, wl_t, bl, wr_t)


def kernel(x, edge_index, W_l, b_l, W_r):
    src = edge_index[0]
    dst = edge_index[1]
    zr = jnp.zeros((NP, DIN), jnp.float32)
    zc = jnp.zeros((NP,), jnp.float32)
    psum, pcnt = _sc_aggregate(x, src, dst, zr, zc)
    return _dense(psum, pcnt, x, W_l.T, b_l.reshape(1, DOUT), W_r.T)


# speedup vs baseline: 14.9926x; 1.1258x over previous
import functools

import jax
import jax.numpy as jnp
from jax import lax
from jax.experimental import pallas as pl
from jax.experimental.pallas import tpu as pltpu
from jax.experimental.pallas import tpu_sc as plsc

N = 10000
NP = 10240
E = 320000
DIN = 128
DOUT = 256

NC = 2
NS = 16
NW = NC * NS
EPW = E // NW
CH = 40
NCHUNK = EPW // CH
RR = 6
LG = RR - 2
SS = RR + 2
RPT = NP // NS


def _sc_aggregate(x, src, dst, zrows, zcnt):
    mesh = plsc.VectorSubcoreMesh(
        core_axis_name="c", subcore_axis_name="s", num_cores=NC, num_subcores=NS
    )

    @functools.partial(
        pl.kernel,
        out_type=(
            jax.ShapeDtypeStruct((NC, NP, DIN), jnp.float32),
            jax.ShapeDtypeStruct((NC, NP), jnp.float32),
        ),
        mesh=mesh,
        scratch_types=(
            pltpu.VMEM_SHARED((NP, DIN), jnp.float32),
            pltpu.VMEM_SHARED((NP,), jnp.float32),
            pltpu.VMEM((SS, CH), jnp.int32),
            pltpu.VMEM((SS, CH), jnp.int32),
            pltpu.VMEM((RR, CH, DIN), jnp.float32),
            pltpu.VMEM((CH,), jnp.float32),
            pltpu.SemaphoreType.DMA((RR,)),
            pltpu.SemaphoreType.DMA((RR,)),
            pltpu.SemaphoreType.DMA((SS,)),
        ),
    )
    def k(x_hbm, src_hbm, dst_hbm, zr_hbm, zc_hbm, psum_hbm, pcnt_hbm,
          acc, cnt, srcix, dstix, rows, ones, gsem, ssem, isem):
        cid = lax.axis_index("c")
        sid = lax.axis_index("s")
        wid = sid * NC + cid
        base = wid * EPW

        r0 = pl.multiple_of(sid * RPT, 128)
        pltpu.sync_copy(zr_hbm.at[pl.ds(r0, RPT)], acc.at[pl.ds(r0, RPT)])
        pltpu.sync_copy(zc_hbm.at[pl.ds(r0, RPT)], cnt.at[pl.ds(r0, RPT)])

        for i in range(CH // 16):
            ones[pl.ds(i * 16, 16)] = jnp.ones((16,), jnp.float32)

        plsc.subcore_barrier()

        def load_idx(j, slot):
            pltpu.async_copy(src_hbm.at[pl.ds(base + j * CH, CH)],
                             srcix.at[slot], isem.at[slot])
            pltpu.async_copy(dst_hbm.at[pl.ds(base + j * CH, CH)],
                             dstix.at[slot], isem.at[slot])

        def wait_idx(slot):
            pltpu.make_async_copy(src_hbm.at[pl.ds(0, CH)],
                                  srcix.at[slot], isem.at[slot]).wait()
            pltpu.make_async_copy(dst_hbm.at[pl.ds(0, CH)],
                                  dstix.at[slot], isem.at[slot]).wait()

        def wait_scatter(slot):
            pltpu.make_async_copy(rows.at[slot], acc.at[dstix.at[0]],
                                  ssem.at[slot]).wait()
            pltpu.make_async_copy(ones, cnt.at[dstix.at[0]],
                                  ssem.at[slot]).wait()

        for j in range(LG + 2):
            load_idx(j, j)
        for j in range(LG):
            wait_idx(j)
            pltpu.async_copy(x_hbm.at[srcix.at[j]], rows.at[j], gsem.at[j])

        def body(j, _):
            br = lax.rem(j, RR)
            si = lax.rem(j, SS)
            pltpu.make_async_copy(
                x_hbm.at[srcix.at[si]], rows.at[br], gsem.at[br]
            ).wait()

            @pl.when(j + LG < NCHUNK)
            def _():
                nbr = lax.rem(j + LG, RR)
                nsi = lax.rem(j + LG, SS)

                @pl.when(j >= 2)
                def _():
                    wait_scatter(nbr)

                wait_idx(nsi)
                pltpu.async_copy(
                    x_hbm.at[srcix.at[nsi]], rows.at[nbr], gsem.at[nbr]
                )

            pltpu.async_copy(rows.at[br], acc.at[dstix.at[si]],
                             ssem.at[br], add=True)
            pltpu.async_copy(ones, cnt.at[dstix.at[si]],
                             ssem.at[br], add=True)

            @pl.when(j + LG + 2 < NCHUNK)
            def _():
                load_idx(j + LG + 2, lax.rem(j + LG + 2, SS))

            return ()

        lax.fori_loop(0, NCHUNK, body, (), unroll=False)

        for j in range(NCHUNK - RR, NCHUNK):
            wait_scatter(j % RR)

        plsc.subcore_barrier()

        pltpu.sync_copy(acc.at[pl.ds(r0, RPT)], psum_hbm.at[cid, pl.ds(r0, RPT)])
        pltpu.sync_copy(cnt.at[pl.ds(r0, RPT)], pcnt_hbm.at[cid, pl.ds(r0, RPT)])

    return k(x, src, dst, zrows, zcnt)


BR = 1024


def _dense_body(ps_ref, pc_ref, x_ref, wl_ref, bl_ref, wr_ref, o_ref):
    s = ps_ref[0] + ps_ref[1]
    c = pc_ref[0] + pc_ref[1]
    inv = 1.0 / jnp.clip(c, 1.0, None)
    mean = s * inv[:, None]
    hi = jax.lax.Precision.HIGHEST
    o_ref[...] = (
        jnp.dot(mean, wl_ref[...], precision=hi)
        + jnp.dot(x_ref[...], wr_ref[...], precision=hi)
        + bl_ref[...]
    )


def _dense(psum, pcnt, x, wl_t, bl, wr_t):
    grid = (pl.cdiv(N, BR),)
    return pl.pallas_call(
        _dense_body,
        grid=grid,
        in_specs=[
            pl.BlockSpec((NC, BR, DIN), lambda i: (0, i, 0)),
            pl.BlockSpec((NC, BR), lambda i: (0, i)),
            pl.BlockSpec((BR, DIN), lambda i: (i, 0)),
            pl.BlockSpec((DIN, DOUT), lambda i: (0, 0)),
            pl.BlockSpec((1, DOUT), lambda i: (0, 0)),
            pl.BlockSpec((DIN, DOUT), lambda i: (0, 0)),
        ],
        out_specs=pl.BlockSpec((BR, DOUT), lambda i: (i, 0)),
        out_shape=jax.ShapeDtypeStruct((N, DOUT), jnp.float32),
    )(psum, pcnt, x, wl_t, bl, wr_t)


def kernel(x, edge_index, W_l, b_l, W_r):
    src = edge_index[0]
    dst = edge_index[1]
    zr = jnp.zeros((NP, DIN), jnp.float32)
    zc = jnp.zeros((NP,), jnp.float32)
    psum, pcnt = _sc_aggregate(x, src, dst, zr, zc)
    return _dense(psum, pcnt, x, W_l.T, b_l.reshape(1, DOUT), W_r.T)


# speedup vs baseline: 15.2099x; 1.0145x over previous
import functools

import jax
import jax.numpy as jnp
from jax import lax
from jax.experimental import pallas as pl
from jax.experimental.pallas import tpu as pltpu
from jax.experimental.pallas import tpu_sc as plsc

N = 10000
NP = 10240
E = 320000
DIN = 128
DOUT = 256

NC = 2
NS = 16
NW = NC * NS
EPW = E // NW
CH = 80
NCHUNK = EPW // CH
RR = 4
LG = 3
SS = 6
RPT = NP // NS


def _sc_aggregate(x, src, dst, zrows, zcnt):
    mesh = plsc.VectorSubcoreMesh(
        core_axis_name="c", subcore_axis_name="s", num_cores=NC, num_subcores=NS
    )

    @functools.partial(
        pl.kernel,
        out_type=(
            jax.ShapeDtypeStruct((NC, NP, DIN), jnp.float32),
            jax.ShapeDtypeStruct((NC, NP), jnp.float32),
        ),
        mesh=mesh,
        scratch_types=(
            pltpu.VMEM_SHARED((NP, DIN), jnp.float32),
            pltpu.VMEM_SHARED((NP,), jnp.float32),
            pltpu.VMEM((SS, CH), jnp.int32),
            pltpu.VMEM((SS, CH), jnp.int32),
            pltpu.VMEM((RR, CH, DIN), jnp.float32),
            pltpu.VMEM((CH,), jnp.float32),
            pltpu.SemaphoreType.DMA((RR,)),
            pltpu.SemaphoreType.DMA((RR,)),
            pltpu.SemaphoreType.DMA((SS,)),
        ),
    )
    def k(x_hbm, src_hbm, dst_hbm, zr_hbm, zc_hbm, psum_hbm, pcnt_hbm,
          acc, cnt, srcix, dstix, rows, ones, gsem, ssem, isem):
        cid = lax.axis_index("c")
        sid = lax.axis_index("s")
        wid = sid * NC + cid
        base = wid * EPW

        r0 = pl.multiple_of(sid * RPT, 128)
        pltpu.sync_copy(zr_hbm.at[pl.ds(r0, RPT)], acc.at[pl.ds(r0, RPT)])
        pltpu.sync_copy(zc_hbm.at[pl.ds(r0, RPT)], cnt.at[pl.ds(r0, RPT)])

        for i in range(CH // 16):
            ones[pl.ds(i * 16, 16)] = jnp.ones((16,), jnp.float32)

        plsc.subcore_barrier()

        def load_idx(j, slot):
            pltpu.async_copy(src_hbm.at[pl.ds(base + j * CH, CH)],
                             srcix.at[slot], isem.at[slot])
            pltpu.async_copy(dst_hbm.at[pl.ds(base + j * CH, CH)],
                             dstix.at[slot], isem.at[slot])

        def wait_idx(slot):
            pltpu.make_async_copy(src_hbm.at[pl.ds(0, CH)],
                                  srcix.at[slot], isem.at[slot]).wait()
            pltpu.make_async_copy(dst_hbm.at[pl.ds(0, CH)],
                                  dstix.at[slot], isem.at[slot]).wait()

        def wait_scatter(slot):
            pltpu.make_async_copy(rows.at[slot], acc.at[dstix.at[0]],
                                  ssem.at[slot]).wait()
            pltpu.make_async_copy(ones, cnt.at[dstix.at[0]],
                                  ssem.at[slot]).wait()

        for j in range(LG + 2):
            load_idx(j, j)
        for j in range(LG):
            wait_idx(j)
            pltpu.async_copy(x_hbm.at[srcix.at[j]], rows.at[j], gsem.at[j])

        def body(j, _):
            br = lax.rem(j, RR)
            si = lax.rem(j, SS)
            pltpu.make_async_copy(
                x_hbm.at[srcix.at[si]], rows.at[br], gsem.at[br]
            ).wait()

            @pl.when(j + LG < NCHUNK)
            def _():
                nbr = lax.rem(j + LG, RR)
                nsi = lax.rem(j + LG, SS)

                @pl.when(j >= 1)
                def _():
                    wait_scatter(nbr)

                wait_idx(nsi)
                pltpu.async_copy(
                    x_hbm.at[srcix.at[nsi]], rows.at[nbr], gsem.at[nbr]
                )

            pltpu.async_copy(rows.at[br], acc.at[dstix.at[si]],
                             ssem.at[br], add=True)
            pltpu.async_copy(ones, cnt.at[dstix.at[si]],
                             ssem.at[br], add=True)

            @pl.when(j + LG + 2 < NCHUNK)
            def _():
                load_idx(j + LG + 2, lax.rem(j + LG + 2, SS))

            return ()

        lax.fori_loop(0, NCHUNK, body, (), unroll=False)

        for j in range(NCHUNK - RR, NCHUNK):
            wait_scatter(j % RR)

        plsc.subcore_barrier()

        pltpu.sync_copy(acc.at[pl.ds(r0, RPT)], psum_hbm.at[cid, pl.ds(r0, RPT)])
        pltpu.sync_copy(cnt.at[pl.ds(r0, RPT)], pcnt_hbm.at[cid, pl.ds(r0, RPT)])

    return k(x, src, dst, zrows, zcnt)


BR = 1024


def _dense_body(ps_ref, pc_ref, x_ref, wl_ref, bl_ref, wr_ref, o_ref):
    s = ps_ref[0] + ps_ref[1]
    c = pc_ref[0] + pc_ref[1]
    inv = 1.0 / jnp.clip(c, 1.0, None)
    mean = s * inv[:, None]
    hi = jax.lax.Precision.HIGHEST
    o_ref[...] = (
        jnp.dot(mean, wl_ref[...], precision=hi)
        + jnp.dot(x_ref[...], wr_ref[...], precision=hi)
        + bl_ref[...]
    )


def _dense(psum, pcnt, x, wl_t, bl, wr_t):
    grid = (pl.cdiv(N, BR),)
    return pl.pallas_call(
        _dense_body,
        grid=grid,
        in_specs=[
            pl.BlockSpec((NC, BR, DIN), lambda i: (0, i, 0)),
            pl.BlockSpec((NC, BR), lambda i: (0, i)),
            pl.BlockSpec((BR, DIN), lambda i: (i, 0)),
            pl.BlockSpec((DIN, DOUT), lambda i: (0, 0)),
            pl.BlockSpec((1, DOUT), lambda i: (0, 0)),
            pl.BlockSpec((DIN, DOUT), lambda i: (0, 0)),
        ],
        out_specs=pl.BlockSpec((BR, DOUT), lambda i: (i, 0)),
        out_shape=jax.ShapeDtypeStruct((N, DOUT), jnp.float32),
    )(psum, pcnt, x, wl_t, bl, wr_t)


def kernel(x, edge_index, W_l, b_l, W_r):
    src = edge_index[0]
    dst = edge_index[1]
    zr = jnp.zeros((NP, DIN), jnp.float32)
    zc = jnp.zeros((NP,), jnp.float32)
    psum, pcnt = _sc_aggregate(x, src, dst, zr, zc)
    return _dense(psum, pcnt, x, W_l.T, b_l.reshape(1, DOUT), W_r.T)


# speedup vs baseline: 15.9360x; 1.0477x over previous
import functools

import jax
import jax.numpy as jnp
from jax import lax
from jax.experimental import pallas as pl
from jax.experimental.pallas import tpu as pltpu
from jax.experimental.pallas import tpu_sc as plsc

N = 10000
NP = 10240
E = 320000
DIN = 128
DOUT = 256

NC = 2
NS = 16
NW = NC * NS
EPW = E // NW
CH = 80
NCHUNK = EPW // CH
RR = 4
LG = 3
SS = 6
RPT = NP // NS


def _sc_aggregate(x, src, dst):
    mesh = plsc.VectorSubcoreMesh(
        core_axis_name="c", subcore_axis_name="s", num_cores=NC, num_subcores=NS
    )

    @functools.partial(
        pl.kernel,
        out_type=(
            jax.ShapeDtypeStruct((NC, NP, DIN), jnp.float32),
            jax.ShapeDtypeStruct((NC, NP), jnp.float32),
        ),
        mesh=mesh,
        scratch_types=(
            pltpu.VMEM_SHARED((NP, DIN), jnp.float32),
            pltpu.VMEM_SHARED((NP,), jnp.float32),
            pltpu.VMEM((SS, CH), jnp.int32),
            pltpu.VMEM((SS, CH), jnp.int32),
            pltpu.VMEM((RR, CH, DIN), jnp.float32),
            pltpu.VMEM((CH,), jnp.float32),
            pltpu.VMEM((RPT,), jnp.float32),
            pltpu.SemaphoreType.DMA((RR,)),
            pltpu.SemaphoreType.DMA((RR,)),
            pltpu.SemaphoreType.DMA((SS,)),
        ),
    )
    def k(x_hbm, src_hbm, dst_hbm, psum_hbm, pcnt_hbm,
          acc, cnt, srcix, dstix, rows, ones, zrow, gsem, ssem, isem):
        cid = lax.axis_index("c")
        sid = lax.axis_index("s")
        wid = sid * NC + cid
        base = wid * EPW

        for i in range(CH // 16):
            ones[pl.ds(i * 16, 16)] = jnp.ones((16,), jnp.float32)
        for i in range(RPT // 16):
            zrow[pl.ds(i * 16, 16)] = jnp.zeros((16,), jnp.float32)

        def zfill(r, _):
            def zcol(c, _):
                rows[0, r, pl.ds(c * 16, 16)] = jnp.zeros((16,), jnp.float32)
                return ()
            lax.fori_loop(0, DIN // 16, zcol, ())
            return ()

        lax.fori_loop(0, CH, zfill, ())

        r0 = pl.multiple_of(sid * RPT, 128)
        for i in range(RPT // CH):
            pltpu.sync_copy(rows.at[0], acc.at[pl.ds(r0 + i * CH, CH)])
        pltpu.sync_copy(zrow, cnt.at[pl.ds(r0, RPT)])

        plsc.subcore_barrier()

        def load_idx(j, slot):
            pltpu.async_copy(src_hbm.at[pl.ds(base + j * CH, CH)],
                             srcix.at[slot], isem.at[slot])
            pltpu.async_copy(dst_hbm.at[pl.ds(base + j * CH, CH)],
                             dstix.at[slot], isem.at[slot])

        def wait_idx(slot):
            pltpu.make_async_copy(src_hbm.at[pl.ds(0, CH)],
                                  srcix.at[slot], isem.at[slot]).wait()
            pltpu.make_async_copy(dst_hbm.at[pl.ds(0, CH)],
                                  dstix.at[slot], isem.at[slot]).wait()

        def wait_scatter(slot):
            pltpu.make_async_copy(rows.at[slot], acc.at[dstix.at[0]],
                                  ssem.at[slot]).wait()
            pltpu.make_async_copy(ones, cnt.at[dstix.at[0]],
                                  ssem.at[slot]).wait()

        for j in range(LG + 2):
            load_idx(j, j)
        for j in range(LG):
            wait_idx(j)
            pltpu.async_copy(x_hbm.at[srcix.at[j]], rows.at[j], gsem.at[j])

        def body(j, _):
            br = lax.rem(j, RR)
            si = lax.rem(j, SS)
            pltpu.make_async_copy(
                x_hbm.at[srcix.at[si]], rows.at[br], gsem.at[br]
            ).wait()

            @pl.when(j + LG < NCHUNK)
            def _():
                nbr = lax.rem(j + LG, RR)
                nsi = lax.rem(j + LG, SS)

                @pl.when(j >= 1)
                def _():
                    wait_scatter(nbr)

                wait_idx(nsi)
                pltpu.async_copy(
                    x_hbm.at[srcix.at[nsi]], rows.at[nbr], gsem.at[nbr]
                )

            pltpu.async_copy(rows.at[br], acc.at[dstix.at[si]],
                             ssem.at[br], add=True)
            pltpu.async_copy(ones, cnt.at[dstix.at[si]],
                             ssem.at[br], add=True)

            @pl.when(j + LG + 2 < NCHUNK)
            def _():
                load_idx(j + LG + 2, lax.rem(j + LG + 2, SS))

            return ()

        lax.fori_loop(0, NCHUNK, body, (), unroll=False)

        for j in range(NCHUNK - RR, NCHUNK):
            wait_scatter(j % RR)

        plsc.subcore_barrier()

        pltpu.sync_copy(acc.at[pl.ds(r0, RPT)], psum_hbm.at[cid, pl.ds(r0, RPT)])
        pltpu.sync_copy(cnt.at[pl.ds(r0, RPT)], pcnt_hbm.at[cid, pl.ds(r0, RPT)])

    return k(x, src, dst)


BR = 1024


def _dense_body(ps_ref, pc_ref, x_ref, wl_ref, bl_ref, wr_ref, o_ref):
    s = ps_ref[0] + ps_ref[1]
    c = pc_ref[0] + pc_ref[1]
    inv = 1.0 / jnp.clip(c, 1.0, None)
    mean = s * inv[:, None]
    hi = jax.lax.Precision.HIGHEST
    o_ref[...] = (
        jnp.dot(mean, wl_ref[...], precision=hi)
        + jnp.dot(x_ref[...], wr_ref[...], precision=hi)
        + bl_ref[...]
    )


def _dense(psum, pcnt, x, wl_t, bl, wr_t):
    grid = (pl.cdiv(N, BR),)
    return pl.pallas_call(
        _dense_body,
        grid=grid,
        in_specs=[
            pl.BlockSpec((NC, BR, DIN), lambda i: (0, i, 0)),
            pl.BlockSpec((NC, BR), lambda i: (0, i)),
            pl.BlockSpec((BR, DIN), lambda i: (i, 0)),
            pl.BlockSpec((DIN, DOUT), lambda i: (0, 0)),
            pl.BlockSpec((1, DOUT), lambda i: (0, 0)),
            pl.BlockSpec((DIN, DOUT), lambda i: (0, 0)),
        ],
        out_specs=pl.BlockSpec((BR, DOUT), lambda i: (i, 0)),
        out_shape=jax.ShapeDtypeStruct((N, DOUT), jnp.float32),
    )(psum, pcnt, x, wl_t, bl, wr_t)


def kernel(x, edge_index, W_l, b_l, W_r):
    src = edge_index[0]
    dst = edge_index[1]
    psum, pcnt = _sc_aggregate(x, src, dst)
    return _dense(psum, pcnt, x, W_l.T, b_l.reshape(1, DOUT), W_r.T)


# speedup vs baseline: 16.2538x; 1.0199x over previous
import functools

import jax
import jax.numpy as jnp
from jax import lax
from jax.experimental import pallas as pl
from jax.experimental.pallas import tpu as pltpu
from jax.experimental.pallas import tpu_sc as plsc

N = 10000
NP = 10240
E = 320000
DIN = 128
DOUT = 256

NC = 2
NS = 16
NW = NC * NS
EPW = E // NW
CH = 80
NCHUNK = EPW // CH
RR = 4
LG = 3
SS = 6
RPT = NP // NS


def _sc_aggregate(x, src, dst):
    mesh = plsc.VectorSubcoreMesh(
        core_axis_name="c", subcore_axis_name="s", num_cores=NC, num_subcores=NS
    )

    @functools.partial(
        pl.kernel,
        out_type=(
            jax.ShapeDtypeStruct((NC, NP, DIN), jnp.float32),
            jax.ShapeDtypeStruct((NC, NP), jnp.float32),
        ),
        mesh=mesh,
        scratch_types=(
            pltpu.VMEM_SHARED((NP, DIN), jnp.float32),
            pltpu.VMEM_SHARED((NP,), jnp.float32),
            pltpu.VMEM((SS, CH), jnp.int32),
            pltpu.VMEM((SS, CH), jnp.int32),
            pltpu.VMEM((RR, CH, DIN), jnp.float32),
            pltpu.VMEM((CH,), jnp.float32),
            pltpu.VMEM((RPT,), jnp.float32),
            pltpu.SemaphoreType.DMA((RR,)),
            pltpu.SemaphoreType.DMA((RR,)),
            pltpu.SemaphoreType.DMA((SS,)),
        ),
    )
    def k(x_hbm, src_hbm, dst_hbm, psum_hbm, pcnt_hbm,
          acc, cnt, srcix, dstix, rows, ones, zrow, gsem, ssem, isem):
        cid = lax.axis_index("c")
        sid = lax.axis_index("s")
        wid = sid * NC + cid
        base = wid * EPW

        for i in range(CH // 16):
            ones[pl.ds(i * 16, 16)] = jnp.ones((16,), jnp.float32)
        for i in range(RPT // 16):
            zrow[pl.ds(i * 16, 16)] = jnp.zeros((16,), jnp.float32)

        def zfill(r, _):
            def zcol(c, _):
                rows[0, r, pl.ds(c * 16, 16)] = jnp.zeros((16,), jnp.float32)
                return ()
            lax.fori_loop(0, DIN // 16, zcol, ())
            return ()

        lax.fori_loop(0, CH, zfill, ())

        r0 = pl.multiple_of(sid * RPT, 128)
        for i in range(RPT // CH):
            pltpu.sync_copy(rows.at[0], acc.at[pl.ds(r0 + i * CH, CH)])
        pltpu.sync_copy(zrow, cnt.at[pl.ds(r0, RPT)])

        plsc.subcore_barrier()

        def load_idx(j, slot):
            pltpu.async_copy(src_hbm.at[pl.ds(base + j * CH, CH)],
                             srcix.at[slot], isem.at[slot])
            pltpu.async_copy(dst_hbm.at[pl.ds(base + j * CH, CH)],
                             dstix.at[slot], isem.at[slot])

        def wait_idx(slot):
            pltpu.make_async_copy(src_hbm.at[pl.ds(0, CH)],
                                  srcix.at[slot], isem.at[slot]).wait()
            pltpu.make_async_copy(dst_hbm.at[pl.ds(0, CH)],
                                  dstix.at[slot], isem.at[slot]).wait()

        def wait_scatter(slot):
            pltpu.make_async_copy(rows.at[slot], acc.at[dstix.at[0]],
                                  ssem.at[slot]).wait()
            pltpu.make_async_copy(ones, cnt.at[dstix.at[0]],
                                  ssem.at[slot]).wait()

        for j in range(LG + 2):
            load_idx(j, j)
        for j in range(LG):
            wait_idx(j)
            pltpu.async_copy(x_hbm.at[srcix.at[j]], rows.at[j], gsem.at[j])

        def body(j, _):
            br = lax.rem(j, RR)
            si = lax.rem(j, SS)
            pltpu.make_async_copy(
                x_hbm.at[srcix.at[si]], rows.at[br], gsem.at[br]
            ).wait()

            @pl.when(j + LG < NCHUNK)
            def _():
                nbr = lax.rem(j + LG, RR)
                nsi = lax.rem(j + LG, SS)

                @pl.when(j >= 1)
                def _():
                    wait_scatter(nbr)

                wait_idx(nsi)
                pltpu.async_copy(
                    x_hbm.at[srcix.at[nsi]], rows.at[nbr], gsem.at[nbr]
                )

            pltpu.async_copy(rows.at[br], acc.at[dstix.at[si]],
                             ssem.at[br], add=True)
            pltpu.async_copy(ones, cnt.at[dstix.at[si]],
                             ssem.at[br], add=True)

            @pl.when(j + LG + 2 < NCHUNK)
            def _():
                load_idx(j + LG + 2, lax.rem(j + LG + 2, SS))

            return ()

        lax.fori_loop(0, NCHUNK, body, (), unroll=False)

        for j in range(NCHUNK - RR, NCHUNK):
            wait_scatter(j % RR)

        plsc.subcore_barrier()

        pltpu.sync_copy(acc.at[pl.ds(r0, RPT)], psum_hbm.at[cid, pl.ds(r0, RPT)])
        pltpu.sync_copy(cnt.at[pl.ds(r0, RPT)], pcnt_hbm.at[cid, pl.ds(r0, RPT)])

    return k(x, src, dst)


BR = 1024


def _dense_a_body(x_ref, wr_ref, bl_ref, o_ref):
    hi = jax.lax.Precision.HIGHEST
    o_ref[...] = jnp.dot(x_ref[...], wr_ref[...], precision=hi) + bl_ref[...]


def _dense_a(x, wr_t, bl):
    grid = (pl.cdiv(N, BR),)
    return pl.pallas_call(
        _dense_a_body,
        grid=grid,
        in_specs=[
            pl.BlockSpec((BR, DIN), lambda i: (i, 0)),
            pl.BlockSpec((DIN, DOUT), lambda i: (0, 0)),
            pl.BlockSpec((1, DOUT), lambda i: (0, 0)),
        ],
        out_specs=pl.BlockSpec((BR, DOUT), lambda i: (i, 0)),
        out_shape=jax.ShapeDtypeStruct((N, DOUT), jnp.float32),
    )(x, wr_t, bl)


def _dense_b_body(ps_ref, pc_ref, a_ref, wl_ref, o_ref):
    s = ps_ref[0] + ps_ref[1]
    c = pc_ref[0] + pc_ref[1]
    inv = 1.0 / jnp.clip(c, 1.0, None)
    mean = s * inv[:, None]
    hi = jax.lax.Precision.HIGHEST
    o_ref[...] = jnp.dot(mean, wl_ref[...], precision=hi) + a_ref[...]


def _dense_b(psum, pcnt, a, wl_t):
    grid = (pl.cdiv(N, BR),)
    return pl.pallas_call(
        _dense_b_body,
        grid=grid,
        in_specs=[
            pl.BlockSpec((NC, BR, DIN), lambda i: (0, i, 0)),
            pl.BlockSpec((NC, BR), lambda i: (0, i)),
            pl.BlockSpec((BR, DOUT), lambda i: (i, 0)),
            pl.BlockSpec((DIN, DOUT), lambda i: (0, 0)),
        ],
        out_specs=pl.BlockSpec((BR, DOUT), lambda i: (i, 0)),
        out_shape=jax.ShapeDtypeStruct((N, DOUT), jnp.float32),
    )(psum, pcnt, a, wl_t)


def kernel(x, edge_index, W_l, b_l, W_r):
    src = edge_index[0]
    dst = edge_index[1]
    psum, pcnt = _sc_aggregate(x, src, dst)
    a = _dense_a(x, W_r.T, b_l.reshape(1, DOUT))
    return _dense_b(psum, pcnt, a, W_l.T)


# speedup vs baseline: 16.3895x; 1.0083x over previous
import functools

import jax
import jax.numpy as jnp
from jax import lax
from jax.experimental import pallas as pl
from jax.experimental.pallas import tpu as pltpu
from jax.experimental.pallas import tpu_sc as plsc

N = 10000
NP = 10240
E = 320000
DIN = 128
DOUT = 256

NC = 2
NS = 16
NW = NC * NS
EPW = E // NW
CH = 80
NCHUNK = EPW // CH
RR = 4
LG = 3
SS = 6
RPT = NP // NS


def _sc_aggregate(x, src, dst):
    mesh = plsc.VectorSubcoreMesh(
        core_axis_name="c", subcore_axis_name="s", num_cores=NC, num_subcores=NS
    )

    @functools.partial(
        pl.kernel,
        out_type=(
            jax.ShapeDtypeStruct((NC, NP, DIN), jnp.float32),
            jax.ShapeDtypeStruct((NC, NP), jnp.float32),
        ),
        mesh=mesh,
        scratch_types=(
            pltpu.VMEM_SHARED((NP, DIN), jnp.float32),
            pltpu.VMEM_SHARED((NP,), jnp.float32),
            pltpu.VMEM((SS, CH), jnp.int32),
            pltpu.VMEM((SS, CH), jnp.int32),
            pltpu.VMEM((RR, CH, DIN), jnp.float32),
            pltpu.VMEM((CH,), jnp.float32),
            pltpu.VMEM((RPT,), jnp.float32),
            pltpu.SemaphoreType.DMA((RR,)),
            pltpu.SemaphoreType.DMA((RR,)),
            pltpu.SemaphoreType.DMA((SS,)),
        ),
    )
    def k(x_hbm, src_hbm, dst_hbm, psum_hbm, pcnt_hbm,
          acc, cnt, srcix, dstix, rows, ones, zrow, gsem, ssem, isem):
        cid = lax.axis_index("c")
        sid = lax.axis_index("s")
        wid = sid * NC + cid
        base = wid * EPW

        for i in range(CH // 16):
            ones[pl.ds(i * 16, 16)] = jnp.ones((16,), jnp.float32)
        for i in range(RPT // 16):
            zrow[pl.ds(i * 16, 16)] = jnp.zeros((16,), jnp.float32)

        def zfill(r, _):
            def zcol(c, _):
                rows[0, r, pl.ds(c * 16, 16)] = jnp.zeros((16,), jnp.float32)
                return ()
            lax.fori_loop(0, DIN // 16, zcol, ())
            return ()

        lax.fori_loop(0, CH, zfill, ())

        r0 = pl.multiple_of(sid * RPT, 128)
        for i in range(RPT // CH):
            pltpu.sync_copy(rows.at[0], acc.at[pl.ds(r0 + i * CH, CH)])
        pltpu.sync_copy(zrow, cnt.at[pl.ds(r0, RPT)])

        plsc.subcore_barrier()

        def load_idx(j, slot):
            pltpu.async_copy(src_hbm.at[pl.ds(base + j * CH, CH)],
                             srcix.at[slot], isem.at[slot])
            pltpu.async_copy(dst_hbm.at[pl.ds(base + j * CH, CH)],
                             dstix.at[slot], isem.at[slot])

        def wait_idx(slot):
            pltpu.make_async_copy(src_hbm.at[pl.ds(0, CH)],
                                  srcix.at[slot], isem.at[slot]).wait()
            pltpu.make_async_copy(dst_hbm.at[pl.ds(0, CH)],
                                  dstix.at[slot], isem.at[slot]).wait()

        def wait_scatter(slot):
            pltpu.make_async_copy(rows.at[slot], acc.at[dstix.at[0]],
                                  ssem.at[slot]).wait()
            pltpu.make_async_copy(ones, cnt.at[dstix.at[0]],
                                  ssem.at[slot]).wait()

        for j in range(LG + 2):
            load_idx(j, j)
        for j in range(LG):
            wait_idx(j)
            pltpu.async_copy(x_hbm.at[srcix.at[j]], rows.at[j], gsem.at[j])

        def body(j, _):
            br = lax.rem(j, RR)
            si = lax.rem(j, SS)
            pltpu.make_async_copy(
                x_hbm.at[srcix.at[si]], rows.at[br], gsem.at[br]
            ).wait()

            @pl.when(j + LG < NCHUNK)
            def _():
                nbr = lax.rem(j + LG, RR)
                nsi = lax.rem(j + LG, SS)

                @pl.when(j >= 1)
                def _():
                    wait_scatter(nbr)

                wait_idx(nsi)
                pltpu.async_copy(
                    x_hbm.at[srcix.at[nsi]], rows.at[nbr], gsem.at[nbr]
                )

            pltpu.async_copy(rows.at[br], acc.at[dstix.at[si]],
                             ssem.at[br], add=True)
            pltpu.async_copy(ones, cnt.at[dstix.at[si]],
                             ssem.at[br], add=True)

            @pl.when(j + LG + 2 < NCHUNK)
            def _():
                load_idx(j + LG + 2, lax.rem(j + LG + 2, SS))

            return ()

        lax.fori_loop(0, NCHUNK, body, (), unroll=False)

        for j in range(NCHUNK - RR, NCHUNK):
            wait_scatter(j % RR)

        plsc.subcore_barrier()

        pltpu.sync_copy(acc.at[pl.ds(r0, RPT)], psum_hbm.at[cid, pl.ds(r0, RPT)])
        pltpu.sync_copy(cnt.at[pl.ds(r0, RPT)], pcnt_hbm.at[cid, pl.ds(r0, RPT)])

    return k(x, src, dst)


BR = 2048


def _dense_a_body(x_ref, wr_ref, bl_ref, o_ref):
    hi = jax.lax.Precision.HIGHEST
    o_ref[...] = jnp.dot(x_ref[...], wr_ref[...], precision=hi) + bl_ref[...]


def _dense_a(x, wr_t, bl):
    grid = (pl.cdiv(N, BR),)
    return pl.pallas_call(
        _dense_a_body,
        grid=grid,
        in_specs=[
            pl.BlockSpec((BR, DIN), lambda i: (i, 0)),
            pl.BlockSpec((DIN, DOUT), lambda i: (0, 0)),
            pl.BlockSpec((1, DOUT), lambda i: (0, 0)),
        ],
        out_specs=pl.BlockSpec((BR, DOUT), lambda i: (i, 0)),
        out_shape=jax.ShapeDtypeStruct((N, DOUT), jnp.float32),
    )(x, wr_t, bl)


def _dense_b_body(ps_ref, pc_ref, a_ref, wl_ref, o_ref):
    s = ps_ref[0] + ps_ref[1]
    c = pc_ref[0] + pc_ref[1]
    inv = 1.0 / jnp.clip(c, 1.0, None)
    mean = s * inv[:, None]
    hi = jax.lax.Precision.HIGHEST
    o_ref[...] = jnp.dot(mean, wl_ref[...], precision=hi) + a_ref[...]


def _dense_b(psum, pcnt, a, wl_t):
    grid = (pl.cdiv(N, BR),)
    return pl.pallas_call(
        _dense_b_body,
        grid=grid,
        in_specs=[
            pl.BlockSpec((NC, BR, DIN), lambda i: (0, i, 0)),
            pl.BlockSpec((NC, BR), lambda i: (0, i)),
            pl.BlockSpec((BR, DOUT), lambda i: (i, 0)),
            pl.BlockSpec((DIN, DOUT), lambda i: (0, 0)),
        ],
        out_specs=pl.BlockSpec((BR, DOUT), lambda i: (i, 0)),
        out_shape=jax.ShapeDtypeStruct((N, DOUT), jnp.float32),
    )(psum, pcnt, a, wl_t)


def kernel(x, edge_index, W_l, b_l, W_r):
    src = edge_index[0]
    dst = edge_index[1]
    psum, pcnt = _sc_aggregate(x, src, dst)
    a = _dense_a(x, W_r.T, b_l.reshape(1, DOUT))
    return _dense_b(psum, pcnt, a, W_l.T)


# speedup vs baseline: 16.6446x; 1.0156x over previous
import functools

import jax
import jax.numpy as jnp
from jax import lax
from jax.experimental import pallas as pl
from jax.experimental.pallas import tpu as pltpu
from jax.experimental.pallas import tpu_sc as plsc

N = 10000
NP = 10240
E = 320000
DIN = 128
DOUT = 256

NC = 2
NS = 16
NW = NC * NS
EPW = E // NW
CH = 80
NCHUNK = EPW // CH
RR = 4
LG = 3
SS = 6
RPT = NP // NS


def _sc_aggregate(x, src, dst):
    mesh = plsc.VectorSubcoreMesh(
        core_axis_name="c", subcore_axis_name="s", num_cores=NC, num_subcores=NS
    )

    @functools.partial(
        pl.kernel,
        out_type=(
            jax.ShapeDtypeStruct((NC, NP, DIN), jnp.float32),
            jax.ShapeDtypeStruct((NC, NP), jnp.float32),
        ),
        mesh=mesh,
        scratch_types=(
            pltpu.VMEM_SHARED((NP, DIN), jnp.float32),
            pltpu.VMEM_SHARED((NP,), jnp.float32),
            pltpu.VMEM((SS, CH), jnp.int32),
            pltpu.VMEM((SS, CH), jnp.int32),
            pltpu.VMEM((RR, CH, DIN), jnp.float32),
            pltpu.VMEM((CH,), jnp.float32),
            pltpu.VMEM((RPT,), jnp.float32),
            pltpu.SemaphoreType.DMA((RR,)),
            pltpu.SemaphoreType.DMA((RR,)),
            pltpu.SemaphoreType.DMA((SS,)),
        ),
    )
    def k(x_hbm, src_hbm, dst_hbm, psum_hbm, pcnt_hbm,
          acc, cnt, srcix, dstix, rows, ones, zrow, gsem, ssem, isem):
        cid = lax.axis_index("c")
        sid = lax.axis_index("s")
        wid = sid * NC + cid
        base = wid * EPW

        def load_idx(j, slot):
            pltpu.async_copy(src_hbm.at[pl.ds(base + j * CH, CH)],
                             srcix.at[slot], isem.at[slot])
            pltpu.async_copy(dst_hbm.at[pl.ds(base + j * CH, CH)],
                             dstix.at[slot], isem.at[slot])

        def wait_idx(slot):
            pltpu.make_async_copy(src_hbm.at[pl.ds(0, CH)],
                                  srcix.at[slot], isem.at[slot]).wait()
            pltpu.make_async_copy(dst_hbm.at[pl.ds(0, CH)],
                                  dstix.at[slot], isem.at[slot]).wait()

        for j in range(LG + 2):
            load_idx(j, j)
        for j in range(1, LG):
            wait_idx(j)
            pltpu.async_copy(x_hbm.at[srcix.at[j]], rows.at[j], gsem.at[j])

        for i in range(CH // 16):
            ones[pl.ds(i * 16, 16)] = jnp.ones((16,), jnp.float32)
        for i in range(RPT // 16):
            zrow[pl.ds(i * 16, 16)] = jnp.zeros((16,), jnp.float32)

        def zfill(r, _):
            def zcol(c, _):
                rows[0, r, pl.ds(c * 16, 16)] = jnp.zeros((16,), jnp.float32)
                return ()
            lax.fori_loop(0, DIN // 16, zcol, ())
            return ()

        lax.fori_loop(0, CH, zfill, ())

        r0 = pl.multiple_of(sid * RPT, 128)
        for i in range(RPT // CH):
            pltpu.sync_copy(rows.at[0], acc.at[pl.ds(r0 + i * CH, CH)])
        pltpu.sync_copy(zrow, cnt.at[pl.ds(r0, RPT)])

        wait_idx(0)
        pltpu.async_copy(x_hbm.at[srcix.at[0]], rows.at[0], gsem.at[0])

        plsc.subcore_barrier()

        def wait_scatter(slot):
            pltpu.make_async_copy(rows.at[slot], acc.at[dstix.at[0]],
                                  ssem.at[slot]).wait()
            pltpu.make_async_copy(ones, cnt.at[dstix.at[0]],
                                  ssem.at[slot]).wait()

        def body(j, _):
            br = lax.rem(j, RR)
            si = lax.rem(j, SS)
            pltpu.make_async_copy(
                x_hbm.at[srcix.at[si]], rows.at[br], gsem.at[br]
            ).wait()

            @pl.when(j + LG < NCHUNK)
            def _():
                nbr = lax.rem(j + LG, RR)
                nsi = lax.rem(j + LG, SS)

                @pl.when(j >= 1)
                def _():
                    wait_scatter(nbr)

                wait_idx(nsi)
                pltpu.async_copy(
                    x_hbm.at[srcix.at[nsi]], rows.at[nbr], gsem.at[nbr]
                )

            pltpu.async_copy(rows.at[br], acc.at[dstix.at[si]],
                             ssem.at[br], add=True)
            pltpu.async_copy(ones, cnt.at[dstix.at[si]],
                             ssem.at[br], add=True)

            @pl.when(j + LG + 2 < NCHUNK)
            def _():
                load_idx(j + LG + 2, lax.rem(j + LG + 2, SS))

            return ()

        lax.fori_loop(0, NCHUNK, body, (), unroll=False)

        for j in range(NCHUNK - RR, NCHUNK):
            wait_scatter(j % RR)

        plsc.subcore_barrier()

        w1 = pltpu.async_copy(acc.at[pl.ds(r0, RPT)],
                              psum_hbm.at[cid, pl.ds(r0, RPT)], gsem.at[0])
        w2 = pltpu.async_copy(cnt.at[pl.ds(r0, RPT)],
                              pcnt_hbm.at[cid, pl.ds(r0, RPT)], gsem.at[0])
        w1.wait()
        w2.wait()

    return k(x, src, dst)


BR = 2048


def _dense_a_body(x_ref, wr_ref, bl_ref, o_ref):
    hi = jax.lax.Precision.HIGHEST
    o_ref[...] = jnp.dot(x_ref[...], wr_ref[...], precision=hi) + bl_ref[...]


def _dense_a(x, wr_t, bl):
    grid = (pl.cdiv(N, BR),)
    return pl.pallas_call(
        _dense_a_body,
        grid=grid,
        in_specs=[
            pl.BlockSpec((BR, DIN), lambda i: (i, 0)),
            pl.BlockSpec((DIN, DOUT), lambda i: (0, 0)),
            pl.BlockSpec((1, DOUT), lambda i: (0, 0)),
        ],
        out_specs=pl.BlockSpec((BR, DOUT), lambda i: (i, 0)),
        out_shape=jax.ShapeDtypeStruct((N, DOUT), jnp.float32),
    )(x, wr_t, bl)


def _dense_b_body(ps_ref, pc_ref, a_ref, wl_ref, o_ref):
    s = ps_ref[0] + ps_ref[1]
    c = pc_ref[0] + pc_ref[1]
    inv = 1.0 / jnp.clip(c, 1.0, None)
    mean = s * inv[:, None]
    hi = jax.lax.Precision.HIGHEST
    o_ref[...] = jnp.dot(mean, wl_ref[...], precision=hi) + a_ref[...]


def _dense_b(psum, pcnt, a, wl_t):
    grid = (pl.cdiv(N, BR),)
    return pl.pallas_call(
        _dense_b_body,
        grid=grid,
        in_specs=[
            pl.BlockSpec((NC, BR, DIN), lambda i: (0, i, 0)),
            pl.BlockSpec((NC, BR), lambda i: (0, i)),
            pl.BlockSpec((BR, DOUT), lambda i: (i, 0)),
            pl.BlockSpec((DIN, DOUT), lambda i: (0, 0)),
        ],
        out_specs=pl.BlockSpec((BR, DOUT), lambda i: (i, 0)),
        out_shape=jax.ShapeDtypeStruct((N, DOUT), jnp.float32),
    )(psum, pcnt, a, wl_t)


def kernel(x, edge_index, W_l, b_l, W_r):
    src = edge_index[0]
    dst = edge_index[1]
    psum, pcnt = _sc_aggregate(x, src, dst)
    a = _dense_a(x, W_r.T, b_l.reshape(1, DOUT))
    return _dense_b(psum, pcnt, a, W_l.T)
